# baseline (speedup 1.0000x reference)
import jax
import jax.numpy as jnp
from jax.experimental import pallas as pl
from jax.experimental.pallas import tpu as pltpu

_EPS = 1e-5


def _zero_halo(pad_ref):
    Hp, Wp, C = pad_ref.shape
    zrow = jnp.zeros((1, Wp, C), jnp.float32)
    zcol = jnp.zeros((Hp, 1, C), jnp.float32)
    pad_ref[0:1, :, :] = zrow
    pad_ref[Hp - 1:Hp, :, :] = zrow
    pad_ref[:, 0:1, :] = zcol
    pad_ref[:, Wp - 1:Wp, :] = zcol


def _im2col_dot(pad_ref, patch_ref, w_ref, H, W, C):
    for kh in range(3):
        for kw in range(3):
            tap = kh * 3 + kw
            win = pad_ref[kh:kh + H, kw:kw + W, :]
            patch_ref[:, tap * C:(tap + 1) * C] = win.reshape(
                H * W, C).astype(jnp.bfloat16)
    return jnp.dot(patch_ref[...], w_ref[...],
                   preferred_element_type=jnp.float32)


def _stat_row(y):
    return jnp.concatenate(
        [jnp.sum(y, axis=0, keepdims=True),
         jnp.sum(y * y, axis=0, keepdims=True)], axis=1)


def _affine(st_ref, g_ref, b_ref, a_ref, count):
    C = g_ref.shape[-1]
    tot = st_ref[...]
    mean = tot[:, :C] / count
    var = jnp.maximum(tot[:, C:] / count - mean * mean, 0.0)
    s = g_ref[...] * jax.lax.rsqrt(var + _EPS)
    t = b_ref[...] - mean * s
    a_ref[0:1, :] = s
    a_ref[1:2, :] = t


def _fused_kernel(x_ref, w1_ref, w2_ref, g1_ref, b1_ref, g2_ref, b2_ref,
                  o_ref, y1_ref, y2_ref, st1_ref, st2_ref, a1_ref, a2_ref,
                  pad_ref, patch_ref, *, count):
    _, H, W, Cin = x_ref.shape
    Cout = o_ref.shape[-1]
    HW = H * W
    p = pl.program_id(0)
    n = pl.program_id(1)

    @pl.when(p == 0)
    def _phase0():
        @pl.when(n == 0)
        def _():
            st1_ref[...] = jnp.zeros_like(st1_ref)
            _zero_halo(pad_ref)
        for i in range(x_ref.shape[0]):
            pad_ref[1:H + 1, 1:W + 1, :] = x_ref[i]
            y = _im2col_dot(pad_ref, patch_ref, w1_ref, H, W, Cin)
            y1_ref[x_ref.shape[0] * n + i] = y.astype(jnp.bfloat16)
            st1_ref[...] += _stat_row(y)

    @pl.when(p == 1)
    def _phase1():
        @pl.when(n == 0)
        def _():
            _affine(st1_ref, g1_ref, b1_ref, a1_ref, count)
            st2_ref[...] = jnp.zeros_like(st2_ref)
        for i in range(x_ref.shape[0]):
            m = x_ref.shape[0] * n + i
            h = jnp.maximum(
                y1_ref[m].astype(jnp.float32) * a1_ref[0:1, :]
                + a1_ref[1:2, :], 0.0)
            pad_ref[1:H + 1, 1:W + 1, :] = h.reshape(H, W, Cout)
            y = _im2col_dot(pad_ref, patch_ref, w2_ref, H, W, Cout)
            y2_ref[m] = y.astype(jnp.bfloat16)
            st2_ref[...] += _stat_row(y)

    @pl.when(p == 2)
    def _phase2():
        @pl.when(n == 0)
        def _():
            _affine(st2_ref, g2_ref, b2_ref, a2_ref, count)
        for i in range(x_ref.shape[0]):
            h = jnp.maximum(
                y2_ref[x_ref.shape[0] * n + i].astype(jnp.float32)
                * a2_ref[0:1, :] + a2_ref[1:2, :], 0.0)
            o_ref[i] = h.reshape(H, W, Cout)


@jax.jit
def _forward(x_nchw, w1, g1, beta1, w2, g2, beta2):
    import functools
    N, Cin, H, W = x_nchw.shape
    Cout = w1.shape[-1]
    x = jnp.transpose(x_nchw, (0, 2, 3, 1))
    w1r = w1.reshape(9 * Cin, Cout).astype(jnp.bfloat16)
    w2r = w2.reshape(9 * Cout, Cout).astype(jnp.bfloat16)
    count = float(N * H * W)

    out_nhwc = pl.pallas_call(
        functools.partial(_fused_kernel, count=count),
        grid=(3, N // 2),
        in_specs=[
            pl.BlockSpec((2, H, W, Cin),
                         lambda p, n: (jnp.where(p == 0, n, 0), 0, 0, 0)),
            pl.BlockSpec((9 * Cin, Cout), lambda p, n: (0, 0)),
            pl.BlockSpec((9 * Cout, Cout), lambda p, n: (0, 0)),
            pl.BlockSpec((1, Cout), lambda p, n: (0, 0)),
            pl.BlockSpec((1, Cout), lambda p, n: (0, 0)),
            pl.BlockSpec((1, Cout), lambda p, n: (0, 0)),
            pl.BlockSpec((1, Cout), lambda p, n: (0, 0)),
        ],
        out_specs=pl.BlockSpec(
            (2, H, W, Cout),
            lambda p, n: (jnp.where(p == 2, n, 0), 0, 0, 0)),
        out_shape=jax.ShapeDtypeStruct((N, H, W, Cout), jnp.float32),
        scratch_shapes=[
            pltpu.VMEM((N, H * W, Cout), jnp.bfloat16),
            pltpu.VMEM((N, H * W, Cout), jnp.bfloat16),
            pltpu.VMEM((1, 2 * Cout), jnp.float32),
            pltpu.VMEM((1, 2 * Cout), jnp.float32),
            pltpu.VMEM((2, Cout), jnp.float32),
            pltpu.VMEM((2, Cout), jnp.float32),
            pltpu.VMEM((H + 2, W + 2, Cin), jnp.float32),
            pltpu.VMEM((H * W, 9 * Cin), jnp.bfloat16),
        ],
        compiler_params=pltpu.CompilerParams(
            dimension_semantics=("arbitrary", "arbitrary"),
            vmem_limit_bytes=60 * 1024 * 1024),
    )(x, w1r, w2r,
      g1.reshape(1, Cout), beta1.reshape(1, Cout),
      g2.reshape(1, Cout), beta2.reshape(1, Cout))

    return jnp.transpose(out_nhwc, (0, 3, 1, 2))


def kernel(x_nchw, w1, b1, g1, beta1, w2, b2, g2, beta2):
    del b1, b2
    return _forward(x_nchw.astype(jnp.float32), w1, g1, beta1, w2, g2, beta2)


# speedup vs baseline: 1.0875x; 1.0875x over previous
import jax
import jax.numpy as jnp
from jax.experimental import pallas as pl
from jax.experimental.pallas import tpu as pltpu

_EPS = 1e-5


def _zero_halo(pad_ref):
    Hp, Wp, C = pad_ref.shape
    zrow = jnp.zeros((1, Wp, C), jnp.float32)
    zcol = jnp.zeros((Hp, 1, C), jnp.float32)
    pad_ref[0:1, :, :] = zrow
    pad_ref[Hp - 1:Hp, :, :] = zrow
    pad_ref[:, 0:1, :] = zcol
    pad_ref[:, Wp - 1:Wp, :] = zcol


def _im2col_dot(pad_ref, patch_ref, w_ref, H, W, C):
    for kh in range(3):
        for kw in range(3):
            tap = kh * 3 + kw
            win = pad_ref[kh:kh + H, kw:kw + W, :]
            patch_ref[:, tap * C:(tap + 1) * C] = win.reshape(
                H * W, C).astype(jnp.bfloat16)
    return jnp.dot(patch_ref[...], w_ref[...],
                   preferred_element_type=jnp.float32)


def _stat_row(y):
    return jnp.concatenate(
        [jnp.sum(y, axis=0, keepdims=True),
         jnp.sum(y * y, axis=0, keepdims=True)], axis=1)


def _affine(st_ref, g_ref, b_ref, a_ref, count):
    C = g_ref.shape[-1]
    tot = st_ref[...]
    mean = tot[:, :C] / count
    var = jnp.maximum(tot[:, C:] / count - mean * mean, 0.0)
    s = g_ref[...] * jax.lax.rsqrt(var + _EPS)
    t = b_ref[...] - mean * s
    a_ref[0:1, :] = s
    a_ref[1:2, :] = t


def _fused_kernel(x_ref, w1_ref, w2_ref, g1_ref, b1_ref, g2_ref, b2_ref,
                  o_ref, y1_ref, y2_ref, st1_ref, st2_ref, a1_ref, a2_ref,
                  pad_ref, patch_ref, *, count):
    _, H, W, Cin = x_ref.shape
    Cout = o_ref.shape[-1]
    HW = H * W
    p = pl.program_id(0)
    n = pl.program_id(1)

    @pl.when(p == 0)
    def _phase0():
        @pl.when(n == 0)
        def _():
            st1_ref[...] = jnp.zeros_like(st1_ref)
            _zero_halo(pad_ref)
        pad_ref[1:H + 1, 1:W + 1, :] = x_ref[0]
        y = _im2col_dot(pad_ref, patch_ref, w1_ref, H, W, Cin)
        y1_ref[n] = y.astype(jnp.bfloat16)
        st1_ref[...] += _stat_row(y)

    @pl.when(p == 1)
    def _phase1():
        @pl.when(n == 0)
        def _():
            _affine(st1_ref, g1_ref, b1_ref, a1_ref, count)
            st2_ref[...] = jnp.zeros_like(st2_ref)
        h = jnp.maximum(
            y1_ref[n].astype(jnp.float32) * a1_ref[0:1, :]
            + a1_ref[1:2, :], 0.0)
        pad_ref[1:H + 1, 1:W + 1, :] = h.reshape(H, W, Cout)
        y = _im2col_dot(pad_ref, patch_ref, w2_ref, H, W, Cout)
        y2_ref[n] = y.astype(jnp.bfloat16)
        st2_ref[...] += _stat_row(y)

    @pl.when(p == 2)
    def _phase2():
        @pl.when(n == 0)
        def _():
            _affine(st2_ref, g2_ref, b2_ref, a2_ref, count)
        h = jnp.maximum(
            y2_ref[n].astype(jnp.float32) * a2_ref[0:1, :]
            + a2_ref[1:2, :], 0.0)
        o_ref[0] = h.reshape(H, W, Cout)


@jax.jit
def _forward(x_nchw, w1, g1, beta1, w2, g2, beta2):
    import functools
    N, Cin, H, W = x_nchw.shape
    Cout = w1.shape[-1]
    x = jnp.transpose(x_nchw, (0, 2, 3, 1))
    w1r = w1.reshape(9 * Cin, Cout).astype(jnp.bfloat16)
    w2r = w2.reshape(9 * Cout, Cout).astype(jnp.bfloat16)
    count = float(N * H * W)

    out_nhwc = pl.pallas_call(
        functools.partial(_fused_kernel, count=count),
        grid=(3, N),
        in_specs=[
            pl.BlockSpec((1, H, W, Cin),
                         lambda p, n: (jnp.where(p == 0, n, 0), 0, 0, 0)),
            pl.BlockSpec((9 * Cin, Cout), lambda p, n: (0, 0)),
            pl.BlockSpec((9 * Cout, Cout), lambda p, n: (0, 0)),
            pl.BlockSpec((1, Cout), lambda p, n: (0, 0)),
            pl.BlockSpec((1, Cout), lambda p, n: (0, 0)),
            pl.BlockSpec((1, Cout), lambda p, n: (0, 0)),
            pl.BlockSpec((1, Cout), lambda p, n: (0, 0)),
        ],
        out_specs=pl.BlockSpec(
            (1, H, W, Cout),
            lambda p, n: (jnp.where(p == 2, n, 0), 0, 0, 0)),
        out_shape=jax.ShapeDtypeStruct((N, H, W, Cout), jnp.float32),
        scratch_shapes=[
            pltpu.VMEM((N, H * W, Cout), jnp.bfloat16),
            pltpu.VMEM((N, H * W, Cout), jnp.bfloat16),
            pltpu.VMEM((1, 2 * Cout), jnp.float32),
            pltpu.VMEM((1, 2 * Cout), jnp.float32),
            pltpu.VMEM((2, Cout), jnp.float32),
            pltpu.VMEM((2, Cout), jnp.float32),
            pltpu.VMEM((H + 2, W + 2, Cin), jnp.float32),
            pltpu.VMEM((H * W, 9 * Cin), jnp.bfloat16),
        ],
        compiler_params=pltpu.CompilerParams(
            dimension_semantics=("arbitrary", "arbitrary"),
            vmem_limit_bytes=60 * 1024 * 1024),
    )(x, w1r, w2r,
      g1.reshape(1, Cout), beta1.reshape(1, Cout),
      g2.reshape(1, Cout), beta2.reshape(1, Cout))

    return jnp.transpose(out_nhwc, (0, 3, 1, 2))


def kernel(x_nchw, w1, b1, g1, beta1, w2, b2, g2, beta2):
    del b1, b2
    return _forward(x_nchw.astype(jnp.float32), w1, g1, beta1, w2, g2, beta2)
